# baseline (speedup 1.0000x reference)
import functools

import jax
import jax.numpy as jnp
from jax import lax
from jax.experimental import pallas as pl
from jax.experimental.pallas import tpu as pltpu
from jax.experimental.pallas import tpu_sc as plsc

N = 8192
BR = 256
C = 4
ROWS = N // C

NC = 2
NS = 16
NW = NC * NS
L = 16
PER_W = ROWS // NW
NCHUNK = PER_W // L
NBUF = min(4, NCHUNK)
EPOCHS = NCHUNK // NBUF


def _tc_argmax_rowsum(f_ref, idx_ref, err_ref):
    i = pl.program_id(0)
    f = f_ref[...]
    rowsum = jnp.sum(f, axis=1, keepdims=True)
    m = jnp.max(f, axis=1, keepdims=True)
    cols = lax.broadcasted_iota(jnp.int32, (BR, N), 1)
    amax = jnp.min(jnp.where(f == m, cols, N), axis=1, keepdims=True)
    idx_ref[...] = amax
    err = jnp.sum(jnp.abs(1.0 - rowsum))

    @pl.when(i == 0)
    def _init():
        err_ref[0, 0] = err

    @pl.when(i != 0)
    def _acc():
        err_ref[0, 0] += err


def _tc_pass(chunk):
    row0 = chunk * (ROWS // BR)
    return pl.pallas_call(
        _tc_argmax_rowsum,
        grid=(ROWS // BR,),
        in_specs=[pl.BlockSpec((BR, N), lambda i: (row0 + i, 0))],
        out_specs=[
            pl.BlockSpec((BR, 1), lambda i: (i, 0)),
            pl.BlockSpec(memory_space=pltpu.SMEM),
        ],
        out_shape=[
            jax.ShapeDtypeStruct((ROWS, 1), jnp.int32),
            jax.ShapeDtypeStruct((1, 1), jnp.float32),
        ],
    )


def _sc_gather_body(row0, dist_hbm, col_hbm, out_hbm, col_v, vals_v, acc_v,
                    *sems):
    wid = lax.axis_index("s") * NC + lax.axis_index("c")
    base = wid * PER_W
    pltpu.sync_copy(col_hbm.at[pl.ds(base, PER_W)], col_v)
    iota16 = lax.iota(jnp.int32, L)

    def fire(k, b):
        v16 = col_v[pl.ds(k * L, L)]
        for j in range(L):
            c = v16[j]
            cb = pl.multiple_of((c >> 7) << 7, 128)
            rt = pl.multiple_of(row0 + base + k * L + (j & ~7), 8)
            pltpu.async_copy(
                dist_hbm.at[pl.ds(rt, 8), pl.ds(cb, 128)],
                vals_v.at[b * L + j], sems[b]
            )

    for b in range(NBUF):
        fire(b, b)

    def epoch(e, acc):
        for b in range(NBUF):
            k = e * NBUF + b
            for j in range(L):
                pltpu.make_async_copy(
                    dist_hbm.at[pl.ds(0, 8), pl.ds(0, 128)],
                    vals_v.at[b * L + j], sems[b]
                ).wait()
            v16 = col_v[pl.ds(k * L, L)]
            for j in range(L):
                lane = jnp.full((L,), v16[j] & 127, jnp.int32)
                for s in range(128 // L):
                    row = vals_v[b * L + j, j & 7, pl.ds(s * L, L)]
                    acc = acc + jnp.where(iota16 + s * L == lane, row, 0.0)

            @pl.when(e + 1 < EPOCHS)
            def _refill():
                fire(k + NBUF, b)
        return acc

    acc = lax.fori_loop(0, EPOCHS, epoch, jnp.zeros((L,), jnp.float32))
    acc_v[...] = acc
    pltpu.sync_copy(acc_v, out_hbm.at[wid])


@functools.lru_cache(maxsize=C)
def _sc_gather(chunk):
    return functools.partial(
        pl.kernel,
        mesh=plsc.VectorSubcoreMesh(core_axis_name="c", subcore_axis_name="s"),
        out_type=jax.ShapeDtypeStruct((NW, L), jnp.float32),
        scratch_types=[
            pltpu.VMEM((PER_W,), jnp.int32),
            pltpu.VMEM((NBUF * L, 8, 128), jnp.float32),
            pltpu.VMEM((L,), jnp.float32),
        ] + [pltpu.SemaphoreType.DMA] * NBUF,
    )(functools.partial(_sc_gather_body, chunk * ROWS))


def kernel(feature, distances, target):
    del target
    total = jnp.float32(0.0)
    for c in range(C):
        col_idx, err2 = _tc_pass(c)(feature)
        partials = _sc_gather(c)(distances, col_idx.reshape(-1))
        total = total + err2[0, 0] + jnp.sum(partials)
    return total


# speedup vs baseline: 1.0762x; 1.0762x over previous
import functools

import jax
import jax.numpy as jnp
from jax import lax
from jax.experimental import pallas as pl
from jax.experimental.pallas import tpu as pltpu
from jax.experimental.pallas import tpu_sc as plsc

N = 8192
BR = 256
CHUNK_ROWS = (6144, 2048)
CHUNK_OFF = (0, 6144)
C = len(CHUNK_ROWS)

NC = 2
NS = 16
NW = NC * NS
L = 16
NBUF = 4


def _tc_argmax_rowsum(f_ref, idx_ref, err_ref):
    i = pl.program_id(0)
    f = f_ref[...]
    rowsum = jnp.sum(f, axis=1, keepdims=True)
    m = jnp.max(f, axis=1, keepdims=True)
    cols = lax.broadcasted_iota(jnp.int32, (BR, N), 1)
    amax = jnp.min(jnp.where(f == m, cols, N), axis=1, keepdims=True)
    idx_ref[...] = amax
    err = jnp.sum(jnp.abs(1.0 - rowsum))

    @pl.when(i == 0)
    def _init():
        err_ref[0, 0] = err

    @pl.when(i != 0)
    def _acc():
        err_ref[0, 0] += err


def _tc_pass(chunk):
    rows = CHUNK_ROWS[chunk]
    row0 = CHUNK_OFF[chunk] // BR
    return pl.pallas_call(
        _tc_argmax_rowsum,
        grid=(rows // BR,),
        in_specs=[pl.BlockSpec((BR, N), lambda i: (row0 + i, 0))],
        out_specs=[
            pl.BlockSpec((BR, 1), lambda i: (i, 0)),
            pl.BlockSpec(memory_space=pltpu.SMEM),
        ],
        out_shape=[
            jax.ShapeDtypeStruct((rows, 1), jnp.int32),
            jax.ShapeDtypeStruct((1, 1), jnp.float32),
        ],
    )


def _sc_gather_body(row0, per_w, epochs, dist_hbm, col_hbm, out_hbm, col_v,
                    vals_v, acc_v, *sems):
    wid = lax.axis_index("s") * NC + lax.axis_index("c")
    base = wid * per_w
    pltpu.sync_copy(col_hbm.at[pl.ds(base, per_w)], col_v)
    iota16 = lax.iota(jnp.int32, L)

    def fire(k, b):
        v16 = col_v[pl.ds(k * L, L)]
        for j in range(L):
            c = v16[j]
            cb = pl.multiple_of((c >> 7) << 7, 128)
            rt = pl.multiple_of(row0 + base + k * L + (j & ~7), 8)
            pltpu.async_copy(
                dist_hbm.at[pl.ds(rt, 8), pl.ds(cb, 128)],
                vals_v.at[b * L + j], sems[b]
            )

    for b in range(NBUF):
        fire(b, b)

    def epoch(e, acc):
        for b in range(NBUF):
            k = e * NBUF + b
            for j in range(L):
                pltpu.make_async_copy(
                    dist_hbm.at[pl.ds(0, 8), pl.ds(0, 128)],
                    vals_v.at[b * L + j], sems[b]
                ).wait()
            v16 = col_v[pl.ds(k * L, L)]
            for j in range(L):
                lane = jnp.full((L,), v16[j] & 127, jnp.int32)
                for s in range(128 // L):
                    row = vals_v[b * L + j, j & 7, pl.ds(s * L, L)]
                    acc = acc + jnp.where(iota16 + s * L == lane, row, 0.0)

            @pl.when(e + 1 < epochs)
            def _refill():
                fire(k + NBUF, b)
        return acc

    acc = lax.fori_loop(0, epochs, epoch, jnp.zeros((L,), jnp.float32))
    acc_v[...] = acc
    pltpu.sync_copy(acc_v, out_hbm.at[wid])


@functools.lru_cache(maxsize=C)
def _sc_gather(chunk):
    per_w = CHUNK_ROWS[chunk] // NW
    epochs = (per_w // L) // NBUF
    return functools.partial(
        pl.kernel,
        mesh=plsc.VectorSubcoreMesh(core_axis_name="c", subcore_axis_name="s"),
        out_type=jax.ShapeDtypeStruct((NW, L), jnp.float32),
        scratch_types=[
            pltpu.VMEM((per_w,), jnp.int32),
            pltpu.VMEM((NBUF * L, 8, 128), jnp.float32),
            pltpu.VMEM((L,), jnp.float32),
        ] + [pltpu.SemaphoreType.DMA] * NBUF,
    )(functools.partial(_sc_gather_body, CHUNK_OFF[chunk], per_w, epochs))


def kernel(feature, distances, target):
    del target
    total = jnp.float32(0.0)
    for c in range(C):
        col_idx, err2 = _tc_pass(c)(feature)
        partials = _sc_gather(c)(distances, col_idx.reshape(-1))
        total = total + err2[0, 0] + jnp.sum(partials)
    return total


# speedup vs baseline: 1.0770x; 1.0007x over previous
import functools

import jax
import jax.numpy as jnp
from jax import lax
from jax.experimental import pallas as pl
from jax.experimental.pallas import tpu as pltpu
from jax.experimental.pallas import tpu_sc as plsc

N = 8192
BR = 256
CHUNK_ROWS = (6144, 2048)
CHUNK_OFF = (0, 6144)
C = len(CHUNK_ROWS)

NC = 2
NS = 16
NW = NC * NS
L = 16
NBUF = 4


def _tc_argmax_rowsum(f_ref, idx_ref, err_ref):
    f = f_ref[...]
    rowsum = jnp.sum(f, axis=1, keepdims=True)
    m = jnp.max(f, axis=1, keepdims=True)
    cols = lax.broadcasted_iota(jnp.int32, (BR, N), 1)
    amax = jnp.min(jnp.where(f == m, cols, N), axis=1, keepdims=True)
    idx_ref[...] = amax
    err_ref[...] = jnp.abs(1.0 - rowsum)


def _tc_pass(chunk):
    rows = CHUNK_ROWS[chunk]
    row0 = CHUNK_OFF[chunk] // BR
    return pl.pallas_call(
        _tc_argmax_rowsum,
        grid=(rows // BR,),
        in_specs=[pl.BlockSpec((BR, N), lambda i: (row0 + i, 0))],
        out_specs=[
            pl.BlockSpec((BR, 1), lambda i: (i, 0)),
            pl.BlockSpec((BR, 1), lambda i: (i, 0)),
        ],
        out_shape=[
            jax.ShapeDtypeStruct((rows, 1), jnp.int32),
            jax.ShapeDtypeStruct((rows, 1), jnp.float32),
        ],
        compiler_params=pltpu.CompilerParams(
            dimension_semantics=("parallel",)),
    )


def _sc_gather_body(row0, per_w, epochs, dist_hbm, col_hbm, out_hbm, col_v,
                    vals_v, acc_v, *sems):
    wid = lax.axis_index("s") * NC + lax.axis_index("c")
    base = wid * per_w
    pltpu.sync_copy(col_hbm.at[pl.ds(base, per_w)], col_v)
    iota16 = lax.iota(jnp.int32, L)

    def fire(k, b):
        v16 = col_v[pl.ds(k * L, L)]
        for j in range(L):
            c = v16[j]
            cb = pl.multiple_of((c >> 7) << 7, 128)
            rt = pl.multiple_of(row0 + base + k * L + (j & ~7), 8)
            pltpu.async_copy(
                dist_hbm.at[pl.ds(rt, 8), pl.ds(cb, 128)],
                vals_v.at[b * L + j], sems[b]
            )

    for b in range(NBUF):
        fire(b, b)

    def epoch(e, acc):
        for b in range(NBUF):
            k = e * NBUF + b
            for j in range(L):
                pltpu.make_async_copy(
                    dist_hbm.at[pl.ds(0, 8), pl.ds(0, 128)],
                    vals_v.at[b * L + j], sems[b]
                ).wait()
            v16 = col_v[pl.ds(k * L, L)]
            for j in range(L):
                lane = jnp.full((L,), v16[j] & 127, jnp.int32)
                for s in range(128 // L):
                    row = vals_v[b * L + j, j & 7, pl.ds(s * L, L)]
                    acc = acc + jnp.where(iota16 + s * L == lane, row, 0.0)

            @pl.when(e + 1 < epochs)
            def _refill():
                fire(k + NBUF, b)
        return acc

    acc = lax.fori_loop(0, epochs, epoch, jnp.zeros((L,), jnp.float32))
    acc_v[...] = acc
    pltpu.sync_copy(acc_v, out_hbm.at[wid])


@functools.lru_cache(maxsize=C)
def _sc_gather(chunk):
    per_w = CHUNK_ROWS[chunk] // NW
    epochs = (per_w // L) // NBUF
    return functools.partial(
        pl.kernel,
        mesh=plsc.VectorSubcoreMesh(core_axis_name="c", subcore_axis_name="s"),
        out_type=jax.ShapeDtypeStruct((NW, L), jnp.float32),
        scratch_types=[
            pltpu.VMEM((per_w,), jnp.int32),
            pltpu.VMEM((NBUF * L, 8, 128), jnp.float32),
            pltpu.VMEM((L,), jnp.float32),
        ] + [pltpu.SemaphoreType.DMA] * NBUF,
    )(functools.partial(_sc_gather_body, CHUNK_OFF[chunk], per_w, epochs))


def kernel(feature, distances, target):
    del target
    total = jnp.float32(0.0)
    for c in range(C):
        col_idx, err2 = _tc_pass(c)(feature)
        partials = _sc_gather(c)(distances, col_idx.reshape(-1))
        total = total + jnp.sum(err2) + jnp.sum(partials)
    return total


# speedup vs baseline: 1.1381x; 1.0567x over previous
import functools

import jax
import jax.numpy as jnp
from jax import lax
from jax.experimental import pallas as pl
from jax.experimental.pallas import tpu as pltpu
from jax.experimental.pallas import tpu_sc as plsc

N = 8192
BR = 256
CHUNK_ROWS = (6144, 2048)
CHUNK_OFF = (0, 6144)
C = len(CHUNK_ROWS)

NC = 2
NS = 16
NW = NC * NS
L = 16
NBUF = 4


def _tc_argmax_rowsum(f_ref, idx_ref, err_ref):
    f = f_ref[...]
    rowsum = jnp.sum(f, axis=1, keepdims=True)
    m = jnp.max(f, axis=1, keepdims=True)
    cols = lax.broadcasted_iota(jnp.int32, (BR, N), 1)
    amax = jnp.min(jnp.where(f == m, cols, N), axis=1, keepdims=True)
    idx_ref[...] = amax
    err_ref[...] = jnp.abs(1.0 - rowsum)


def _tc_pass(chunk):
    rows = CHUNK_ROWS[chunk]
    row0 = CHUNK_OFF[chunk] // BR
    return pl.pallas_call(
        _tc_argmax_rowsum,
        grid=(rows // BR,),
        in_specs=[pl.BlockSpec((BR, N), lambda i: (row0 + i, 0))],
        out_specs=[
            pl.BlockSpec((BR, 1), lambda i: (i, 0)),
            pl.BlockSpec((BR, 1), lambda i: (i, 0)),
        ],
        out_shape=[
            jax.ShapeDtypeStruct((rows, 1), jnp.int32),
            jax.ShapeDtypeStruct((rows, 1), jnp.float32),
        ],
        compiler_params=pltpu.CompilerParams(
            dimension_semantics=("parallel",)),
    )


def _sc_gather_body(row0, per_w, epochs, dist_hbm, col_hbm, out_hbm, col_v,
                    vals_v, acc_v, *sems):
    wid = lax.axis_index("s") * NC + lax.axis_index("c")
    base = wid * per_w
    pltpu.sync_copy(col_hbm.at[pl.ds(base, per_w)], col_v)
    iota16 = lax.iota(jnp.int32, L)

    def fire(k, b):
        v16 = col_v[pl.ds(k * L, L)]
        for j in range(L):
            c = v16[j]
            cb = pl.multiple_of((c >> 7) << 7, 128)
            r = row0 + base + k * L + j
            pltpu.async_copy(
                dist_hbm.at[r, pl.ds(cb, 128)],
                vals_v.at[b * L + j], sems[b]
            )

    for b in range(NBUF):
        fire(b, b)

    def epoch(e, acc):
        for b in range(NBUF):
            k = e * NBUF + b
            for j in range(L):
                pltpu.make_async_copy(
                    dist_hbm.at[0, pl.ds(0, 128)],
                    vals_v.at[b * L + j], sems[b]
                ).wait()
            v16 = col_v[pl.ds(k * L, L)]
            for j in range(L):
                lane = jnp.full((L,), v16[j] & 127, jnp.int32)
                for s in range(128 // L):
                    row = vals_v[b * L + j, pl.ds(s * L, L)]
                    acc = acc + jnp.where(iota16 + s * L == lane, row, 0.0)

            @pl.when(e + 1 < epochs)
            def _refill():
                fire(k + NBUF, b)
        return acc

    acc = lax.fori_loop(0, epochs, epoch, jnp.zeros((L,), jnp.float32))
    acc_v[...] = acc
    pltpu.sync_copy(acc_v, out_hbm.at[wid])


@functools.lru_cache(maxsize=C)
def _sc_gather(chunk):
    per_w = CHUNK_ROWS[chunk] // NW
    epochs = (per_w // L) // NBUF
    return functools.partial(
        pl.kernel,
        mesh=plsc.VectorSubcoreMesh(core_axis_name="c", subcore_axis_name="s"),
        out_type=jax.ShapeDtypeStruct((NW, L), jnp.float32),
        scratch_types=[
            pltpu.VMEM((per_w,), jnp.int32),
            pltpu.VMEM((NBUF * L, 128), jnp.float32),
            pltpu.VMEM((L,), jnp.float32),
        ] + [pltpu.SemaphoreType.DMA] * NBUF,
    )(functools.partial(_sc_gather_body, CHUNK_OFF[chunk], per_w, epochs))


def kernel(feature, distances, target):
    del target
    total = jnp.float32(0.0)
    for c in range(C):
        col_idx, err2 = _tc_pass(c)(feature)
        partials = _sc_gather(c)(distances, col_idx.reshape(-1))
        total = total + jnp.sum(err2) + jnp.sum(partials)
    return total


# speedup vs baseline: 1.1442x; 1.0054x over previous
import functools

import jax
import jax.numpy as jnp
from jax import lax
from jax.experimental import pallas as pl
from jax.experimental.pallas import tpu as pltpu
from jax.experimental.pallas import tpu_sc as plsc

N = 8192
BR = 256
CHUNK_ROWS = (6144, 2048)
CHUNK_OFF = (0, 6144)
C = len(CHUNK_ROWS)

NC = 2
NS = 16
NW = NC * NS
L = 16
NBUF = 4


def _tc_argmax_rowsum(f_ref, idx_ref, err_ref):
    f = f_ref[...]
    rowsum = jnp.sum(f, axis=1, keepdims=True)
    m = jnp.max(f, axis=1, keepdims=True)
    cols = lax.broadcasted_iota(jnp.int32, (BR, N), 1)
    amax = jnp.min(jnp.where(f == m, cols, N), axis=1, keepdims=True)
    idx_ref[...] = amax
    err_ref[...] = jnp.abs(1.0 - rowsum)


def _tc_pass(chunk):
    rows = CHUNK_ROWS[chunk]
    row0 = CHUNK_OFF[chunk] // BR
    return pl.pallas_call(
        _tc_argmax_rowsum,
        grid=(rows // BR,),
        in_specs=[pl.BlockSpec((BR, N), lambda i: (row0 + i, 0))],
        out_specs=[
            pl.BlockSpec((BR, 1), lambda i: (i, 0)),
            pl.BlockSpec((BR, 1), lambda i: (i, 0)),
        ],
        out_shape=[
            jax.ShapeDtypeStruct((rows, 1), jnp.int32),
            jax.ShapeDtypeStruct((rows, 1), jnp.float32),
        ],
        compiler_params=pltpu.CompilerParams(
            dimension_semantics=("parallel",)),
    )


def _sc_gather_body(row0, per_w, epochs, dist_hbm, col_hbm, out_hbm, col_v,
                    vals_v, acc_v, *sems):
    wid = lax.axis_index("s") * NC + lax.axis_index("c")
    base = wid * per_w
    pltpu.sync_copy(col_hbm.at[pl.ds(base, per_w)], col_v)
    iota16 = lax.iota(jnp.int32, L)

    def fire(k, b):
        v16 = col_v[pl.ds(k * L, L)]
        for j in range(L):
            c = v16[j]
            cb = pl.multiple_of((c >> 4) << 4, L)
            r = row0 + base + k * L + j
            pltpu.async_copy(
                dist_hbm.at[r, pl.ds(cb, L)],
                vals_v.at[b * L + j], sems[b]
            )

    for b in range(NBUF):
        fire(b, b)

    def epoch(e, acc):
        for b in range(NBUF):
            k = e * NBUF + b
            for j in range(L):
                pltpu.make_async_copy(
                    dist_hbm.at[0, pl.ds(0, L)],
                    vals_v.at[b * L + j], sems[b]
                ).wait()
            v16 = col_v[pl.ds(k * L, L)]
            for j in range(L):
                lane = jnp.full((L,), v16[j] & (L - 1), jnp.int32)
                acc = acc + jnp.where(iota16 == lane, vals_v[b * L + j], 0.0)

            @pl.when(e + 1 < epochs)
            def _refill():
                fire(k + NBUF, b)
        return acc

    acc = lax.fori_loop(0, epochs, epoch, jnp.zeros((L,), jnp.float32))
    acc_v[...] = acc
    pltpu.sync_copy(acc_v, out_hbm.at[wid])


@functools.lru_cache(maxsize=C)
def _sc_gather(chunk):
    per_w = CHUNK_ROWS[chunk] // NW
    epochs = (per_w // L) // NBUF
    return functools.partial(
        pl.kernel,
        mesh=plsc.VectorSubcoreMesh(core_axis_name="c", subcore_axis_name="s"),
        out_type=jax.ShapeDtypeStruct((NW, L), jnp.float32),
        scratch_types=[
            pltpu.VMEM((per_w,), jnp.int32),
            pltpu.VMEM((NBUF * L, L), jnp.float32),
            pltpu.VMEM((L,), jnp.float32),
        ] + [pltpu.SemaphoreType.DMA] * NBUF,
    )(functools.partial(_sc_gather_body, CHUNK_OFF[chunk], per_w, epochs))


def kernel(feature, distances, target):
    del target
    total = jnp.float32(0.0)
    for c in range(C):
        col_idx, err2 = _tc_pass(c)(feature)
        partials = _sc_gather(c)(distances, col_idx.reshape(-1))
        total = total + jnp.sum(err2) + jnp.sum(partials)
    return total
